# baseline (speedup 1.0000x reference)
import functools

import jax
import jax.numpy as jnp
from jax import lax
from jax.experimental import pallas as pl
from jax.experimental.pallas import tpu as pltpu
from jax.experimental.pallas import tpu_sc as plsc

B, Q, C = 1024, 900, 91
NW = 32
RPW = B // NW
NG = (Q + 15) // 16
LEN_CLS = ((Q * C + 4) + 7) // 8 * 8
LEN_OBJ = ((Q + 4) + 7) // 8 * 8


def _sc_matcher(cls_flat, obj_flat, tgt):
    mesh = plsc.VectorSubcoreMesh(core_axis_name="c", subcore_axis_name="s")

    @functools.partial(
        pl.kernel,
        mesh=mesh,
        compiler_params=pltpu.CompilerParams(needs_layout_passes=False),
        out_type=jax.ShapeDtypeStruct((B,), jnp.int32),
        scratch_types=[
            pltpu.VMEM((LEN_CLS,), jnp.float32),
            pltpu.VMEM((LEN_OBJ,), jnp.float32),
            pltpu.VMEM((RPW,), jnp.int32),
            pltpu.VMEM((RPW,), jnp.int32),
        ],
    )
    def k(cls_hbm, obj_hbm, tgt_hbm, out_hbm, cls_v, obj_v, tgt_v, res_v):
        wid = lax.axis_index("s") * 2 + lax.axis_index("c")
        lane = lax.iota(jnp.int32, 16)
        pltpu.sync_copy(tgt_hbm.at[pl.ds(wid * RPW, RPW)], tgt_v)

        def row_body(j, carry):
            b = wid * RPW + j
            s_cls = b * (Q * C)
            a_cls = pl.multiple_of((s_cls // 8) * 8, 8)
            sh_cls = s_cls - a_cls
            pltpu.sync_copy(cls_hbm.at[pl.ds(a_cls, LEN_CLS)], cls_v)
            s_obj = b * Q
            a_obj = pl.multiple_of((s_obj // 8) * 8, 8)
            sh_obj = s_obj - a_obj
            pltpu.sync_copy(obj_hbm.at[pl.ds(a_obj, LEN_OBJ)], obj_v)
            t_vec = plsc.load_gather(tgt_v, [jnp.full((16,), j, jnp.int32)])

            def grp_body(g, gc):
                best_r, best_q = gc
                qv = jnp.minimum(g * 16, Q - 16) + lane
                base = sh_cls + qv * C
                a0 = jnp.zeros((16,), jnp.float32)
                a1 = jnp.zeros((16,), jnp.float32)
                a2 = jnp.zeros((16,), jnp.float32)
                a3 = jnp.zeros((16,), jnp.float32)
                for c in range(0, C - 3, 4):
                    a0 += jnp.exp(plsc.load_gather(cls_v, [base + c]))
                    a1 += jnp.exp(plsc.load_gather(cls_v, [base + (c + 1)]))
                    a2 += jnp.exp(plsc.load_gather(cls_v, [base + (c + 2)]))
                    a3 += jnp.exp(plsc.load_gather(cls_v, [base + (c + 3)]))
                for c in range(C - C % 4, C):
                    a0 += jnp.exp(plsc.load_gather(cls_v, [base + c]))
                ssum = (a0 + a1) + (a2 + a3)
                gv = plsc.load_gather(cls_v, [base + t_vec])
                ov = plsc.load_gather(obj_v, [sh_obj + qv])
                r = jnp.exp(gv) / ((1.0 + jnp.exp(-ov)) * ssum)
                upd = r > best_r
                return jnp.where(upd, r, best_r), jnp.where(upd, qv, best_q)

            best_r, best_q = lax.fori_loop(
                0, NG, grp_body,
                (jnp.full((16,), -1.0, jnp.float32), jnp.zeros((16,), jnp.int32)),
            )
            m = jnp.max(best_r)
            cand = jnp.where(best_r == m, best_q, jnp.int32(2**30))
            res = jnp.broadcast_to(jnp.min(cand), (16,))
            plsc.store_scatter(
                res_v, [jnp.full((16,), j, jnp.int32)], res, mask=lane == 0
            )
            return carry

        lax.fori_loop(0, RPW, row_body, 0)
        pltpu.sync_copy(res_v, out_hbm.at[pl.ds(wid * RPW, RPW)])

    return k(cls_flat, obj_flat, tgt)


def kernel(class_logits, obj_logits, targets):
    cls_flat = class_logits.reshape(-1)
    obj_flat = obj_logits.reshape(-1)
    tgt = targets.astype(jnp.int32)
    return _sc_matcher(cls_flat, obj_flat, tgt)


# speedup vs baseline: 2.5954x; 2.5954x over previous
import functools

import jax
import jax.numpy as jnp
from jax import lax
from jax.experimental import pallas as pl
from jax.experimental.pallas import tpu as pltpu
from jax.experimental.pallas import tpu_sc as plsc

B, Q, C = 1024, 900, 91
NW = 32
RPW = B // NW
NG = (Q + 15) // 16


def _sc_matcher(class_logits, obj_logits, tgt):
    mesh = plsc.VectorSubcoreMesh(core_axis_name="c", subcore_axis_name="s")

    @functools.partial(
        pl.kernel,
        mesh=mesh,
        compiler_params=pltpu.CompilerParams(needs_layout_passes=False),
        out_type=jax.ShapeDtypeStruct((B,), jnp.int32),
        scratch_types=[
            pltpu.VMEM((Q, C), jnp.float32),
            pltpu.VMEM((Q,), jnp.float32),
            pltpu.VMEM((RPW,), jnp.int32),
            pltpu.VMEM((RPW,), jnp.int32),
        ],
    )
    def k(cls_hbm, obj_hbm, tgt_hbm, out_hbm, cls_v, obj_v, tgt_v, res_v):
        wid = lax.axis_index("s") * 2 + lax.axis_index("c")
        lane = lax.iota(jnp.int32, 16)
        pltpu.sync_copy(tgt_hbm.at[pl.ds(wid * RPW, RPW)], tgt_v)

        def row_body(j, carry):
            b = wid * RPW + j
            pltpu.sync_copy(cls_hbm.at[b], cls_v)
            pltpu.sync_copy(obj_hbm.at[b], obj_v)
            t_vec = plsc.load_gather(tgt_v, [jnp.full((16,), j, jnp.int32)])

            def grp_body(g, gc):
                best_r, best_q = gc
                qv = jnp.minimum(g * 16, Q - 16) + lane
                a0 = jnp.zeros((16,), jnp.float32)
                a1 = jnp.zeros((16,), jnp.float32)
                a2 = jnp.zeros((16,), jnp.float32)
                a3 = jnp.zeros((16,), jnp.float32)
                for c in range(0, C - 3, 4):
                    a0 += jnp.exp(plsc.load_gather(cls_v, [qv, jnp.full((16,), c, jnp.int32)]))
                    a1 += jnp.exp(plsc.load_gather(cls_v, [qv, jnp.full((16,), c + 1, jnp.int32)]))
                    a2 += jnp.exp(plsc.load_gather(cls_v, [qv, jnp.full((16,), c + 2, jnp.int32)]))
                    a3 += jnp.exp(plsc.load_gather(cls_v, [qv, jnp.full((16,), c + 3, jnp.int32)]))
                for c in range(C - C % 4, C):
                    a0 += jnp.exp(plsc.load_gather(cls_v, [qv, jnp.full((16,), c, jnp.int32)]))
                ssum = (a0 + a1) + (a2 + a3)
                gv = plsc.load_gather(cls_v, [qv, t_vec])
                ov = plsc.load_gather(obj_v, [qv])
                r = jnp.exp(gv) / ((1.0 + jnp.exp(-ov)) * ssum)
                upd = r > best_r
                return jnp.where(upd, r, best_r), jnp.where(upd, qv, best_q)

            best_r, best_q = lax.fori_loop(
                0, NG, grp_body,
                (jnp.full((16,), -1.0, jnp.float32), jnp.zeros((16,), jnp.int32)),
            )
            m = jnp.max(best_r)
            cand = jnp.where(best_r == m, best_q, jnp.int32(2**30))
            res = jnp.broadcast_to(jnp.min(cand), (16,))
            plsc.store_scatter(
                res_v, [jnp.full((16,), j, jnp.int32)], res, mask=lane == 0
            )
            return carry

        lax.fori_loop(0, RPW, row_body, 0)
        pltpu.sync_copy(res_v, out_hbm.at[pl.ds(wid * RPW, RPW)])

    return k(class_logits, obj_logits, tgt)


def kernel(class_logits, obj_logits, targets):
    return _sc_matcher(class_logits, obj_logits, targets.astype(jnp.int32))


# speedup vs baseline: 4.8005x; 1.8496x over previous
import functools

import jax
import jax.numpy as jnp
from jax import lax
from jax.experimental import pallas as pl
from jax.experimental.pallas import tpu as pltpu
from jax.experimental.pallas import tpu_sc as plsc

B, Q, C = 1024, 900, 91
NW = 32
RPW = B // NW
NG = (Q + 15) // 16


def _sc_matcher(class_logits, obj_logits, tgt):
    mesh = plsc.VectorSubcoreMesh(core_axis_name="c", subcore_axis_name="s")

    @functools.partial(
        pl.kernel,
        mesh=mesh,
        compiler_params=pltpu.CompilerParams(needs_layout_passes=False),
        out_type=jax.ShapeDtypeStruct((B,), jnp.int32),
        scratch_types=[
            pltpu.VMEM((Q * C,), jnp.float32),
            pltpu.VMEM((Q,), jnp.float32),
            pltpu.VMEM((RPW,), jnp.int32),
            pltpu.VMEM((RPW,), jnp.int32),
        ],
    )
    def k(cls_hbm, obj_hbm, tgt_hbm, out_hbm, cls_v, obj_v, tgt_v, res_v):
        wid = lax.axis_index("s") * 2 + lax.axis_index("c")
        lane = lax.iota(jnp.int32, 16)
        pltpu.sync_copy(tgt_hbm.at[pl.ds(wid * RPW, RPW)], tgt_v)

        def row_body(j, carry):
            b = wid * RPW + j
            pltpu.sync_copy(cls_hbm.at[b], cls_v)
            pltpu.sync_copy(obj_hbm.at[b], obj_v)
            t_vec = plsc.load_gather(tgt_v, [jnp.full((16,), j, jnp.int32)])

            def grp_body(g, gc):
                best_r, best_q = gc
                qv = jnp.minimum(g * 16, Q - 16) + lane
                base = qv * C
                a0 = jnp.zeros((16,), jnp.float32)
                a1 = jnp.zeros((16,), jnp.float32)
                a2 = jnp.zeros((16,), jnp.float32)
                a3 = jnp.zeros((16,), jnp.float32)
                for c in range(0, C - 3, 4):
                    a0 += jnp.exp(plsc.load_gather(cls_v, [base + c]))
                    a1 += jnp.exp(plsc.load_gather(cls_v, [base + (c + 1)]))
                    a2 += jnp.exp(plsc.load_gather(cls_v, [base + (c + 2)]))
                    a3 += jnp.exp(plsc.load_gather(cls_v, [base + (c + 3)]))
                for c in range(C - C % 4, C):
                    a0 += jnp.exp(plsc.load_gather(cls_v, [base + c]))
                ssum = (a0 + a1) + (a2 + a3)
                gv = plsc.load_gather(cls_v, [base + t_vec])
                ov = plsc.load_gather(obj_v, [qv])
                r = jnp.exp(gv) / ((1.0 + jnp.exp(-ov)) * ssum)
                upd = r > best_r
                return jnp.where(upd, r, best_r), jnp.where(upd, qv, best_q)

            best_r, best_q = lax.fori_loop(
                0, NG, grp_body,
                (jnp.full((16,), -1.0, jnp.float32), jnp.zeros((16,), jnp.int32)),
            )
            m = jnp.max(best_r)
            cand = jnp.where(best_r == m, best_q, jnp.int32(2**30))
            res = jnp.broadcast_to(jnp.min(cand), (16,))
            plsc.store_scatter(
                res_v, [jnp.full((16,), j, jnp.int32)], res, mask=lane == 0
            )
            return carry

        lax.fori_loop(0, RPW, row_body, 0)
        pltpu.sync_copy(res_v, out_hbm.at[pl.ds(wid * RPW, RPW)])

    return k(class_logits.reshape(B, Q * C), obj_logits, tgt)


def kernel(class_logits, obj_logits, targets):
    return _sc_matcher(class_logits, obj_logits, targets.astype(jnp.int32))
